# Initial kernel scaffold; baseline (speedup 1.0000x reference)
import jax
import jax.numpy as jnp
from jax.experimental import pallas as pl


def kernel(x, edge_index, W1, b1, W2, b2):
    raise NotImplementedError("write your pallas kernel here")


# speedup vs baseline: 42.3808x; 42.3808x over previous
import functools

import jax
import jax.numpy as jnp
from jax import lax
from jax.experimental import pallas as pl
from jax.experimental.pallas import tpu as pltpu
from jax.experimental.pallas import tpu_sc as plsc

N = 10000
E = 320000
F_IN = 128
H = 300
C = 64

NG = 10240
EP = 327680
CHUNKS = EP // 128
B_W = CHUNKS // 32
ROWS_T = NG // 16

_mesh = plsc.VectorSubcoreMesh(core_axis_name="c", subcore_axis_name="s")


@functools.partial(
    pl.kernel,
    out_type=jax.ShapeDtypeStruct((2, NG), jnp.float32),
    mesh=_mesh,
    scratch_types=[
        pltpu.VMEM((B_W, 128), jnp.int32),
        pltpu.VMEM((128,), jnp.float32),
        pltpu.VMEM((ROWS_T,), jnp.float32),
        pltpu.VMEM_SHARED((NG,), jnp.float32),
        pltpu.SemaphoreType.DMA,
    ],
)
def _sc_degree(dst_hbm, out_hbm, idx_v, ones_v, zero_v, deg_sh, sem):
    cid = lax.axis_index("c")
    sid = lax.axis_index("s")
    wid = sid * 2 + cid
    one16 = jnp.ones((16,), jnp.float32)
    zero16 = jnp.zeros((16,), jnp.float32)
    for i in range(8):
        ones_v[pl.ds(i * 16, 16)] = one16
    for i in range(ROWS_T // 16):
        zero_v[pl.ds(i * 16, 16)] = zero16
    pltpu.sync_copy(zero_v, deg_sh.at[pl.ds(sid * ROWS_T, ROWS_T)])
    pltpu.sync_copy(dst_hbm.at[pl.ds(wid * B_W, B_W)], idx_v)
    plsc.subcore_barrier()
    for g0 in range(0, B_W, 16):
        descs = [
            pltpu.async_copy(ones_v, deg_sh.at[idx_v.at[j]], sem, add=True)
            for j in range(g0, g0 + 16)
        ]
        for d in descs:
            d.wait()
    plsc.subcore_barrier()
    pltpu.sync_copy(
        deg_sh.at[pl.ds(sid * ROWS_T, ROWS_T)],
        out_hbm.at[cid, pl.ds(sid * ROWS_T, ROWS_T)],
    )


@functools.partial(
    pl.kernel,
    out_type=jax.ShapeDtypeStruct((2, NG, C), jnp.float32),
    mesh=_mesh,
    compiler_params=pltpu.CompilerParams(use_tc_tiling_on_sc=False),
    scratch_types=[
        pltpu.VMEM((B_W, 128), jnp.int32),
        pltpu.VMEM((B_W, 128), jnp.int32),
        pltpu.VMEM((8, 128, C), jnp.float32),
        pltpu.SemaphoreType.DMA,
        pltpu.SemaphoreType.DMA,
        pltpu.VMEM_SHARED((NG, C), jnp.float32),
    ],
)
def _sc_scatter(g_hbm, src_hbm, dst_hbm, out_hbm, src_v, dst_v, rows_v,
                gsem, ssem, agg_sh):
    cid = lax.axis_index("c")
    sid = lax.axis_index("s")
    wid = sid * 2 + cid
    pltpu.sync_copy(
        g_hbm.at[pl.ds(sid * ROWS_T, ROWS_T)],
        agg_sh.at[pl.ds(sid * ROWS_T, ROWS_T)],
    )
    pltpu.sync_copy(src_hbm.at[pl.ds(wid * B_W, B_W)], src_v)
    pltpu.sync_copy(dst_hbm.at[pl.ds(wid * B_W, B_W)], dst_v)
    plsc.subcore_barrier()

    NBUF = 4
    NGRP = B_W // NBUF
    gd = [None] * 8
    sd = [None] * 8

    def issue_gathers(grp):
        base = (grp % 2) * NBUF
        for b in range(NBUF):
            gd[base + b] = pltpu.async_copy(
                g_hbm.at[src_v.at[grp * NBUF + b]], rows_v.at[base + b], gsem
            )

    issue_gathers(0)
    for grp in range(NGRP):
        cur = (grp % 2) * NBUF
        for b in range(NBUF):
            gd[cur + b].wait()
        for b in range(NBUF):
            sd[cur + b] = pltpu.async_copy(
                rows_v.at[cur + b],
                agg_sh.at[dst_v.at[grp * NBUF + b]],
                ssem,
                add=True,
            )
        if grp + 1 < NGRP:
            oth = ((grp + 1) % 2) * NBUF
            if grp >= 1:
                for b in range(NBUF):
                    sd[oth + b].wait()
            issue_gathers(grp + 1)
    last = ((NGRP - 1) % 2) * NBUF
    for b in range(NBUF):
        sd[last + b].wait()
    if NGRP >= 2:
        prev = ((NGRP - 2) % 2) * NBUF
        for b in range(NBUF):
            sd[prev + b].wait()

    plsc.subcore_barrier()
    pltpu.sync_copy(
        agg_sh.at[pl.ds(sid * ROWS_T, ROWS_T)],
        out_hbm.at[cid, pl.ds(sid * ROWS_T, ROWS_T)],
    )


_R = 512


def _tc_dense_body(x_ref, w1_ref, b1_ref, w2_ref, deg_ref, g_ref, dinv_ref):
    h = jnp.dot(x_ref[...], w1_ref[...],
                preferred_element_type=jnp.float32,
                precision=lax.Precision.HIGHEST)
    h = jnp.maximum(h + b1_ref[...], 0.0)
    hw = jnp.dot(h, w2_ref[...],
                 preferred_element_type=jnp.float32,
                 precision=lax.Precision.HIGHEST)
    d = deg_ref[...]
    dinv = lax.rsqrt(d[:, 0:1] + d[:, 1:2] + 1.0)
    i = pl.program_id(0)
    rows = lax.broadcasted_iota(jnp.int32, (_R, 1), 0) + i * _R
    g_ref[...] = jnp.where(rows < N, hw * dinv, 0.0)
    dinv_ref[...] = dinv


def _tc_dense(x_p, W1, b1, W2, degT):
    return pl.pallas_call(
        _tc_dense_body,
        grid=(NG // _R,),
        in_specs=[
            pl.BlockSpec((_R, F_IN), lambda i: (i, 0)),
            pl.BlockSpec((F_IN, H), lambda i: (0, 0)),
            pl.BlockSpec((1, H), lambda i: (0, 0)),
            pl.BlockSpec((H, C), lambda i: (0, 0)),
            pl.BlockSpec((_R, 2), lambda i: (i, 0)),
        ],
        out_specs=[
            pl.BlockSpec((_R, C), lambda i: (i, 0)),
            pl.BlockSpec((_R, 1), lambda i: (i, 0)),
        ],
        out_shape=[
            jax.ShapeDtypeStruct((NG, C), jnp.float32),
            jax.ShapeDtypeStruct((NG, 1), jnp.float32),
        ],
    )(x_p, W1, b1, W2, degT)


def _tc_softmax_body(s_ref, g_ref, dinv_ref, b2_ref, out_ref):
    s = s_ref[0] + s_ref[1] - g_ref[...]
    v = jnp.maximum(s * dinv_ref[...] + b2_ref[...], 0.0)
    m = jnp.max(v, axis=1, keepdims=True)
    lse = jnp.log(jnp.sum(jnp.exp(v - m), axis=1, keepdims=True))
    out_ref[...] = v - m - lse


def _tc_softmax(s_part, g, dinv, b2):
    return pl.pallas_call(
        _tc_softmax_body,
        grid=(NG // _R,),
        in_specs=[
            pl.BlockSpec((2, _R, C), lambda i: (0, i, 0)),
            pl.BlockSpec((_R, C), lambda i: (i, 0)),
            pl.BlockSpec((_R, 1), lambda i: (i, 0)),
            pl.BlockSpec((1, C), lambda i: (0, 0)),
        ],
        out_specs=pl.BlockSpec((_R, C), lambda i: (i, 0)),
        out_shape=jax.ShapeDtypeStruct((NG, C), jnp.float32),
    )(s_part, g, dinv, b2)


def kernel(x, edge_index, W1, b1, W2, b2):
    src = edge_index[0]
    dst = edge_index[1]
    pad_e = EP - E
    fill = (N + (jnp.arange(pad_e, dtype=jnp.int32) % (NG - N))).astype(jnp.int32)
    src_p = jnp.concatenate([src, fill]).reshape(CHUNKS, 128)
    dst_p = jnp.concatenate([dst, fill]).reshape(CHUNKS, 128)
    x_p = jnp.pad(x, ((0, NG - N), (0, 0)))

    deg2 = _sc_degree(dst_p)
    degT = jnp.transpose(deg2)
    g, dinv = _tc_dense(x_p, W1, b1.reshape(1, H), W2, degT)
    s_part = _sc_scatter(g, src_p, dst_p)
    out = _tc_softmax(s_part, g, dinv, b2.reshape(1, C))
    return out[:N]
